# baseline (speedup 1.0000x reference)
import functools

import jax
import jax.numpy as jnp
from jax import lax
from jax.experimental import pallas as pl
from jax.experimental.pallas import tpu as pltpu
from jax.experimental.pallas import tpu_sc as plsc

N = 10000
E = 320000
D_IN = 128
D_HID = 256
D_OUT = 128

NC = 2
NS = 16
DQ = 64
CHK = 128
EP = 327680
NPAD = EP - E
NDEAD = 200
NROW = EP // CHK
RPT = NROW // NS
RPT_A = NROW // (NC * NS)
ZR = 200
NSLOT = 4

_mesh = plsc.VectorSubcoreMesh(core_axis_name="c", subcore_axis_name="s")


@functools.partial(
    pl.kernel,
    out_type=jax.ShapeDtypeStruct((NC * 10240,), jnp.float32),
    mesh=_mesh,
    compiler_params=pltpu.CompilerParams(use_tc_tiling_on_sc=False),
    scratch_types=[
        pltpu.VMEM((RPT_A, CHK), jnp.int32),
        pltpu.VMEM((CHK,), jnp.float32),
        pltpu.VMEM((640,), jnp.float32),
        pltpu.VMEM_SHARED((10240,), jnp.float32),
    ],
)
def _deg_kernel(dst_hbm, out_hbm, idx_v, ones_v, zero_v, hist_sh):
    c = lax.axis_index("c")
    s = lax.axis_index("s")
    w = c * NS + s

    pltpu.sync_copy(dst_hbm.at[pl.ds(w * RPT_A, RPT_A)], idx_v)

    def _fill(i, _):
        ones_v[pl.ds(i * 16, 16)] = jnp.ones((16,), jnp.float32)
        return 0
    lax.fori_loop(0, CHK // 16, _fill, 0)

    def _zfill(i, _):
        zero_v[pl.ds(i * 16, 16)] = jnp.zeros((16,), jnp.float32)
        return 0
    lax.fori_loop(0, 640 // 16, _zfill, 0)
    pltpu.sync_copy(zero_v, hist_sh.at[pl.ds(s * 640, 640)])
    plsc.subcore_barrier()

    def _scat(j, _):
        pltpu.sync_copy(ones_v, hist_sh.at[idx_v.at[j]], add=True)
        return 0
    lax.fori_loop(0, RPT_A, _scat, 0)
    plsc.subcore_barrier()

    pltpu.sync_copy(hist_sh.at[pl.ds(s * 640, 640)], zero_v)
    pltpu.sync_copy(zero_v, out_hbm.at[pl.ds(c * 10240 + s * 640, 640)])


def _make_agg_kernel(passes, scale_output):
    nsl = NC * passes
    out_t = (jax.ShapeDtypeStruct((N, NC * DQ), jnp.float32) if scale_output
             else jax.ShapeDtypeStruct((nsl * N, DQ), jnp.float32))

    scratch = [
        pltpu.VMEM((RPT, CHK), jnp.int32),
        pltpu.VMEM((RPT, CHK), jnp.int32),
        pltpu.VMEM((NSLOT, CHK, DQ), jnp.float32),
        pltpu.VMEM((ZR, DQ), jnp.float32),
        pltpu.VMEM((1016,), jnp.float32),
        pltpu.VMEM_SHARED((N + NDEAD, DQ), jnp.float32),
        pltpu.SemaphoreType.DMA,
        pltpu.SemaphoreType.DMA,
    ]

    @functools.partial(
        pl.kernel,
        out_type=out_t,
        mesh=_mesh,
        compiler_params=pltpu.CompilerParams(use_tc_tiling_on_sc=False),
        scratch_types=scratch,
    )
    def _agg(tbl_hbm, src_hbm, dst_hbm, dinv_hbm, out_hbm,
             sidx_v, didx_v, gbuf, zbuf, dinv_v, acc_sh, sem, sem_s):
        c = lax.axis_index("c")
        s = lax.axis_index("s")

        pltpu.sync_copy(dst_hbm.at[pl.ds(s * RPT, RPT)], didx_v)
        pltpu.sync_copy(src_hbm.at[pl.ds(s * RPT, RPT)], sidx_v)

        for p in range(passes):
            q = c * passes + p
            qoff = q * N if p == 0 else N

            def _off(r, _):
                for k in range(CHK // 16):
                    v = sidx_v[r, pl.ds(k * 16, 16)]
                    sidx_v[r, pl.ds(k * 16, 16)] = v + qoff
                return 0
            lax.fori_loop(0, RPT, _off, 0)

            @pl.when(s < 10)
            def _zero():
                def _zfill(t, _):
                    r = t // (DQ // 16)
                    qq = t % (DQ // 16)
                    zbuf[r, pl.ds(qq * 16, 16)] = jnp.zeros((16,), jnp.float32)
                    return 0
                lax.fori_loop(0, ZR * (DQ // 16), _zfill, 0)
                for k in range(1000 // ZR):
                    pltpu.sync_copy(zbuf,
                                    acc_sh.at[pl.ds(s * 1000 + k * ZR, ZR)])

            @pl.when(s == 10)
            def _zero_dead():
                def _zfill(t, _):
                    r = t // (DQ // 16)
                    qq = t % (DQ // 16)
                    zbuf[r, pl.ds(qq * 16, 16)] = jnp.zeros((16,), jnp.float32)
                    return 0
                lax.fori_loop(0, ZR * (DQ // 16), _zfill, 0)
                pltpu.sync_copy(zbuf, acc_sh.at[pl.ds(N, NDEAD)])
            plsc.subcore_barrier()

            for b in range(2):
                pltpu.async_copy(tbl_hbm.at[sidx_v.at[b]], gbuf.at[b], sem)

            def _edgeN(jj, _):
                for b in range(NSLOT):
                    j = jj * NSLOT + b
                    pltpu.make_async_copy(tbl_hbm.at[sidx_v.at[j]],
                                          gbuf.at[b], sem).wait()
                    pltpu.async_copy(gbuf.at[b], acc_sh.at[didx_v.at[j]],
                                    sem_s, add=True)
                    bp = (b + 2) % NSLOT
                    jm = jnp.maximum(j - 2, 0)

                    @pl.when(j >= 2)
                    def _wait_s():
                        pltpu.make_async_copy(gbuf.at[bp],
                                              acc_sh.at[didx_v.at[jm]],
                                              sem_s).wait()
                    jn = jnp.minimum(j + 2, RPT - 1)

                    @pl.when(j + 2 < RPT)
                    def _fire():
                        pltpu.async_copy(tbl_hbm.at[sidx_v.at[jn]],
                                         gbuf.at[bp], sem)
                return 0
            lax.fori_loop(0, RPT // NSLOT, _edgeN, 0)
            for j in range(RPT - 2, RPT):
                b = j % NSLOT
                pltpu.make_async_copy(gbuf.at[b], acc_sh.at[didx_v.at[j]],
                                      sem_s).wait()
            plsc.subcore_barrier()

            if not scale_output:
                @pl.when(s < 10)
                def _drain():
                    def _blk(k, _):
                        base = s * 1000 + k * ZR
                        pltpu.sync_copy(acc_sh.at[pl.ds(base, ZR)], zbuf)
                        pltpu.sync_copy(zbuf,
                                        out_hbm.at[pl.ds(q * N + base, ZR)])
                        return 0
                    lax.fori_loop(0, 1000 // ZR, _blk, 0)
            else:
                @pl.when(s < 10)
                def _drain():
                    pltpu.sync_copy(dinv_hbm.at[pl.ds(s * 1000, 1000)],
                                    dinv_v.at[pl.ds(0, 1000)])

                    def _blk(k, _):
                        base = s * 1000 + k * ZR
                        pltpu.sync_copy(acc_sh.at[pl.ds(base, ZR)], zbuf)

                        def _row(r, _):
                            dv = dinv_v[pl.ds(k * ZR + r, 16)][0]
                            for qq in range(DQ // 16):
                                v = zbuf[r, pl.ds(qq * 16, 16)]
                                zbuf[r, pl.ds(qq * 16, 16)] = v * dv
                            return 0
                        lax.fori_loop(0, ZR, _row, 0)
                        pltpu.sync_copy(
                            zbuf,
                            out_hbm.at[pl.ds(base, ZR), pl.ds(c * DQ, DQ)])
                        return 0
                    lax.fori_loop(0, 1000 // ZR, _blk, 0)
            if p + 1 < passes:
                plsc.subcore_barrier()

    return _agg


_agg1 = _make_agg_kernel(passes=D_HID // DQ // NC, scale_output=False)
_agg2 = _make_agg_kernel(passes=D_OUT // DQ // NC, scale_output=True)


RB = 1000
NQ1 = D_HID // DQ


def _mm1_body(hist_ref, x_ref, w_ref, b_ref, out_ref, dinv_ref):
    deg = jnp.maximum(hist_ref[0, 0, :] + hist_ref[0, 1, :], 1.0)
    dinv = lax.rsqrt(deg)
    dinv_ref[0, 0] = dinv
    h = jnp.dot(x_ref[...], w_ref[0], preferred_element_type=jnp.float32)
    out_ref[...] = (h + b_ref[0, 0][None, :]) * dinv[:, None]


def _mm1(hist3, x, W1r, b1r):
    grid = (N // RB, NQ1)
    return pl.pallas_call(
        _mm1_body,
        grid=grid,
        in_specs=[
            pl.BlockSpec((1, NC, RB), lambda i, j: (i, 0, 0)),
            pl.BlockSpec((RB, D_IN), lambda i, j: (i, 0)),
            pl.BlockSpec((1, D_IN, DQ), lambda i, j: (j, 0, 0)),
            pl.BlockSpec((1, 1, DQ), lambda i, j: (j, 0, 0)),
        ],
        out_specs=[
            pl.BlockSpec((RB, DQ), lambda i, j: (j * (N // RB) + i, 0)),
            pl.BlockSpec((1, 1, RB), lambda i, j: (i, 0, 0)),
        ],
        out_shape=[
            jax.ShapeDtypeStruct((NQ1 * N, DQ), jnp.float32),
            jax.ShapeDtypeStruct((N // RB, 1, RB), jnp.float32),
        ],
    )(hist3, x, W1r, b1r)


def _mm2_body(a0, a1, a2, a3, dinv_ref, w_ref, b_ref, out_ref):
    dinv = dinv_ref[0, 0]
    h = b_ref[0, 0][None, :] * jnp.ones((RB, 1), jnp.float32)
    for q, a in enumerate((a0, a1, a2, a3)):
        t = jnp.maximum(a[...] * dinv[:, None], 0.0)
        h = h + jnp.dot(t, w_ref[q, 0], preferred_element_type=jnp.float32)
    out_ref[...] = h * dinv[:, None]


def _mm2(raw1, dinv3, W2r, b2r):
    grid = (N // RB, NC)
    nb = N // RB
    return pl.pallas_call(
        _mm2_body,
        grid=grid,
        in_specs=[
            pl.BlockSpec((RB, DQ), lambda i, j: (0 * nb + i, 0)),
            pl.BlockSpec((RB, DQ), lambda i, j: (1 * nb + i, 0)),
            pl.BlockSpec((RB, DQ), lambda i, j: (2 * nb + i, 0)),
            pl.BlockSpec((RB, DQ), lambda i, j: (3 * nb + i, 0)),
            pl.BlockSpec((1, 1, RB), lambda i, j: (i, 0, 0)),
            pl.BlockSpec((NQ1, 1, DQ, DQ), lambda i, j: (0, j, 0, 0)),
            pl.BlockSpec((1, 1, DQ), lambda i, j: (j, 0, 0)),
        ],
        out_specs=pl.BlockSpec((RB, DQ), lambda i, j: (j * nb + i, 0)),
        out_shape=jax.ShapeDtypeStruct((NC * N, DQ), jnp.float32),
    )(raw1, raw1, raw1, raw1, dinv3, W2r, b2r)


def kernel(x, edge_index, W1, b1, W2, b2):
    src = edge_index[0].astype(jnp.int32)
    dst = edge_index[1].astype(jnp.int32)
    pad_src = (jnp.arange(NPAD, dtype=jnp.int32) * 97) % N
    pad_dst = N + (jnp.arange(NPAD, dtype=jnp.int32) % NDEAD)
    srcp = jnp.concatenate([src, pad_src]).reshape(NROW, CHK)
    dstp = jnp.concatenate([dst, pad_dst]).reshape(NROW, CHK)
    W1r = W1.reshape(D_IN, NQ1, DQ).transpose(1, 0, 2)
    b1r = b1.reshape(NQ1, 1, DQ)
    W2r = W2.reshape(NQ1, DQ, NC, DQ).transpose(0, 2, 1, 3)
    b2r = b2.reshape(NC, 1, DQ)

    hist = _deg_kernel(dstp).reshape(NC, 10240)[:, :N]
    hist3 = hist.reshape(NC, N // RB, RB).transpose(1, 0, 2)
    h1s, dinv3 = _mm1(hist3, x, W1r, b1r)
    dinv = dinv3.reshape(N)
    raw1 = _agg1(h1s, srcp, dstp, dinv)
    h2s = _mm2(raw1, dinv3, W2r, b2r)
    return _agg2(h2s, srcp, dstp, dinv)


# speedup vs baseline: 1.0963x; 1.0963x over previous
import functools

import jax
import jax.numpy as jnp
from jax import lax
from jax.experimental import pallas as pl
from jax.experimental.pallas import tpu as pltpu
from jax.experimental.pallas import tpu_sc as plsc

N = 10000
E = 320000
D_IN = 128
D_HID = 256
D_OUT = 128

NC = 2
NS = 16
DQ = 64
CHK = 128
EP = 327680
NPAD = EP - E
NDEAD = 200
NROW = EP // CHK
RPT = NROW // NS
RPT_A = NROW // (NC * NS)
ZR = 200
NSLOT = 4

_mesh = plsc.VectorSubcoreMesh(core_axis_name="c", subcore_axis_name="s")


@functools.partial(
    pl.kernel,
    out_type=jax.ShapeDtypeStruct((NC * 10240,), jnp.float32),
    mesh=_mesh,
    compiler_params=pltpu.CompilerParams(use_tc_tiling_on_sc=False),
    scratch_types=[
        pltpu.VMEM((RPT_A, CHK), jnp.int32),
        pltpu.VMEM((CHK,), jnp.float32),
        pltpu.VMEM((640,), jnp.float32),
        pltpu.VMEM_SHARED((10240,), jnp.float32),
    ],
)
def _deg_kernel(dst_hbm, out_hbm, idx_v, ones_v, zero_v, hist_sh):
    c = lax.axis_index("c")
    s = lax.axis_index("s")
    w = c * NS + s

    pltpu.sync_copy(dst_hbm.at[pl.ds(w * RPT_A, RPT_A)], idx_v)

    def _fill(i, _):
        ones_v[pl.ds(i * 16, 16)] = jnp.ones((16,), jnp.float32)
        return 0
    lax.fori_loop(0, CHK // 16, _fill, 0)

    def _zfill(i, _):
        zero_v[pl.ds(i * 16, 16)] = jnp.zeros((16,), jnp.float32)
        return 0
    lax.fori_loop(0, 640 // 16, _zfill, 0)
    pltpu.sync_copy(zero_v, hist_sh.at[pl.ds(s * 640, 640)])
    plsc.subcore_barrier()

    def _scat(j, _):
        pltpu.sync_copy(ones_v, hist_sh.at[idx_v.at[j]], add=True)
        return 0
    lax.fori_loop(0, RPT_A, _scat, 0)
    plsc.subcore_barrier()

    pltpu.sync_copy(hist_sh.at[pl.ds(s * 640, 640)], zero_v)
    pltpu.sync_copy(zero_v, out_hbm.at[pl.ds(c * 10240 + s * 640, 640)])


def _make_agg_kernel(passes, scale_output):
    nsl = NC * passes
    out_t = (jax.ShapeDtypeStruct((N, NC * DQ), jnp.float32) if scale_output
             else jax.ShapeDtypeStruct((nsl * N, DQ), jnp.float32))

    scratch = [
        pltpu.VMEM((RPT, CHK), jnp.int32),
        pltpu.VMEM((RPT, CHK), jnp.int32),
        pltpu.VMEM((NSLOT, CHK, DQ), jnp.float32),
        pltpu.VMEM((ZR, DQ), jnp.float32),
        pltpu.VMEM((1016,), jnp.float32),
        pltpu.VMEM_SHARED((N + NDEAD, DQ), jnp.float32),
        pltpu.SemaphoreType.DMA,
    ]

    @functools.partial(
        pl.kernel,
        out_type=out_t,
        mesh=_mesh,
        compiler_params=pltpu.CompilerParams(use_tc_tiling_on_sc=False),
        scratch_types=scratch,
    )
    def _agg(tbl_hbm, src_hbm, dst_hbm, dinv_hbm, out_hbm,
             sidx_v, didx_v, gbuf, zbuf, dinv_v, acc_sh, sem):
        c = lax.axis_index("c")
        s = lax.axis_index("s")

        pltpu.sync_copy(dst_hbm.at[pl.ds(s * RPT, RPT)], didx_v)
        pltpu.sync_copy(src_hbm.at[pl.ds(s * RPT, RPT)], sidx_v)

        for p in range(passes):
            q = c * passes + p
            qoff = q * N if p == 0 else N

            def _off(r, _):
                for k in range(CHK // 16):
                    v = sidx_v[r, pl.ds(k * 16, 16)]
                    sidx_v[r, pl.ds(k * 16, 16)] = v + qoff
                return 0
            lax.fori_loop(0, RPT, _off, 0)

            @pl.when(s < 10)
            def _zero():
                def _zfill(t, _):
                    r = t // (DQ // 16)
                    qq = t % (DQ // 16)
                    zbuf[r, pl.ds(qq * 16, 16)] = jnp.zeros((16,), jnp.float32)
                    return 0
                lax.fori_loop(0, ZR * (DQ // 16), _zfill, 0)
                for k in range(1000 // ZR):
                    pltpu.sync_copy(zbuf,
                                    acc_sh.at[pl.ds(s * 1000 + k * ZR, ZR)])

            @pl.when(s == 10)
            def _zero_dead():
                def _zfill(t, _):
                    r = t // (DQ // 16)
                    qq = t % (DQ // 16)
                    zbuf[r, pl.ds(qq * 16, 16)] = jnp.zeros((16,), jnp.float32)
                    return 0
                lax.fori_loop(0, ZR * (DQ // 16), _zfill, 0)
                pltpu.sync_copy(zbuf, acc_sh.at[pl.ds(N, NDEAD)])
            plsc.subcore_barrier()

            for b in range(NSLOT - 1):
                pltpu.async_copy(tbl_hbm.at[sidx_v.at[b]], gbuf.at[b], sem)

            def _edgeN(jj, _):
                for b in range(NSLOT):
                    j = jj * NSLOT + b
                    pltpu.make_async_copy(tbl_hbm.at[sidx_v.at[j]],
                                          gbuf.at[b], sem).wait()
                    jn = jnp.minimum(j + NSLOT - 1, RPT - 1)

                    @pl.when(j + NSLOT - 1 < RPT)
                    def _fire():
                        pltpu.async_copy(tbl_hbm.at[sidx_v.at[jn]],
                                         gbuf.at[(b + NSLOT - 1) % NSLOT], sem)
                    pltpu.sync_copy(gbuf.at[b], acc_sh.at[didx_v.at[j]],
                                    add=True)
                return 0
            lax.fori_loop(0, RPT // NSLOT, _edgeN, 0)
            plsc.subcore_barrier()

            if not scale_output:
                @pl.when(s < 10)
                def _drain():
                    def _blk(k, _):
                        base = s * 1000 + k * ZR
                        pltpu.sync_copy(acc_sh.at[pl.ds(base, ZR)], zbuf)
                        pltpu.sync_copy(zbuf,
                                        out_hbm.at[pl.ds(q * N + base, ZR)])
                        return 0
                    lax.fori_loop(0, 1000 // ZR, _blk, 0)
            else:
                @pl.when(s < 10)
                def _drain():
                    pltpu.sync_copy(dinv_hbm.at[pl.ds(s * 1000, 1000)],
                                    dinv_v.at[pl.ds(0, 1000)])

                    def _blk(k, _):
                        base = s * 1000 + k * ZR
                        pltpu.sync_copy(acc_sh.at[pl.ds(base, ZR)], zbuf)

                        def _row16(g, _):
                            r0 = g * 16
                            dvv = dinv_v[pl.ds(k * ZR + r0, 16)]
                            for rr in range(16):
                                dv = dvv[rr]
                                for qq in range(DQ // 16):
                                    v = zbuf[r0 + rr, pl.ds(qq * 16, 16)]
                                    zbuf[r0 + rr, pl.ds(qq * 16, 16)] = v * dv
                            return 0
                        lax.fori_loop(0, ZR // 16, _row16, 0)
                        r0 = (ZR // 16) * 16
                        dvv = dinv_v[pl.ds(k * ZR + r0, 16)]
                        for rr in range(ZR % 16):
                            dv = dvv[rr]
                            for qq in range(DQ // 16):
                                v = zbuf[r0 + rr, pl.ds(qq * 16, 16)]
                                zbuf[r0 + rr, pl.ds(qq * 16, 16)] = v * dv
                        pltpu.sync_copy(
                            zbuf,
                            out_hbm.at[pl.ds(base, ZR), pl.ds(c * DQ, DQ)])
                        return 0
                    lax.fori_loop(0, 1000 // ZR, _blk, 0)
            if p + 1 < passes:
                plsc.subcore_barrier()

    return _agg


_agg1 = _make_agg_kernel(passes=D_HID // DQ // NC, scale_output=False)
_agg2 = _make_agg_kernel(passes=D_OUT // DQ // NC, scale_output=True)


RB = 1000
NQ1 = D_HID // DQ


def _mm1_body(hist_ref, x_ref, w_ref, b_ref, out_ref, dinv_ref):
    deg = jnp.maximum(hist_ref[0, 0, :] + hist_ref[0, 1, :], 1.0)
    dinv = lax.rsqrt(deg)
    dinv_ref[0, 0] = dinv
    h = jnp.dot(x_ref[...], w_ref[0], preferred_element_type=jnp.float32)
    out_ref[...] = (h + b_ref[0, 0][None, :]) * dinv[:, None]


def _mm1(hist3, x, W1r, b1r):
    grid = (N // RB, NQ1)
    return pl.pallas_call(
        _mm1_body,
        grid=grid,
        in_specs=[
            pl.BlockSpec((1, NC, RB), lambda i, j: (i, 0, 0)),
            pl.BlockSpec((RB, D_IN), lambda i, j: (i, 0)),
            pl.BlockSpec((1, D_IN, DQ), lambda i, j: (j, 0, 0)),
            pl.BlockSpec((1, 1, DQ), lambda i, j: (j, 0, 0)),
        ],
        out_specs=[
            pl.BlockSpec((RB, DQ), lambda i, j: (j * (N // RB) + i, 0)),
            pl.BlockSpec((1, 1, RB), lambda i, j: (i, 0, 0)),
        ],
        out_shape=[
            jax.ShapeDtypeStruct((NQ1 * N, DQ), jnp.float32),
            jax.ShapeDtypeStruct((N // RB, 1, RB), jnp.float32),
        ],
    )(hist3, x, W1r, b1r)


def _mm2_body(a0, a1, a2, a3, dinv_ref, w_ref, b_ref, out_ref):
    dinv = dinv_ref[0, 0]
    h = b_ref[0, 0][None, :] * jnp.ones((RB, 1), jnp.float32)
    for q, a in enumerate((a0, a1, a2, a3)):
        t = jnp.maximum(a[...] * dinv[:, None], 0.0)
        h = h + jnp.dot(t, w_ref[q, 0], preferred_element_type=jnp.float32)
    out_ref[...] = h * dinv[:, None]


def _mm2(raw1, dinv3, W2r, b2r):
    grid = (N // RB, NC)
    nb = N // RB
    return pl.pallas_call(
        _mm2_body,
        grid=grid,
        in_specs=[
            pl.BlockSpec((RB, DQ), lambda i, j: (0 * nb + i, 0)),
            pl.BlockSpec((RB, DQ), lambda i, j: (1 * nb + i, 0)),
            pl.BlockSpec((RB, DQ), lambda i, j: (2 * nb + i, 0)),
            pl.BlockSpec((RB, DQ), lambda i, j: (3 * nb + i, 0)),
            pl.BlockSpec((1, 1, RB), lambda i, j: (i, 0, 0)),
            pl.BlockSpec((NQ1, 1, DQ, DQ), lambda i, j: (0, j, 0, 0)),
            pl.BlockSpec((1, 1, DQ), lambda i, j: (j, 0, 0)),
        ],
        out_specs=pl.BlockSpec((RB, DQ), lambda i, j: (j * nb + i, 0)),
        out_shape=jax.ShapeDtypeStruct((NC * N, DQ), jnp.float32),
    )(raw1, raw1, raw1, raw1, dinv3, W2r, b2r)


def kernel(x, edge_index, W1, b1, W2, b2):
    src = edge_index[0].astype(jnp.int32)
    dst = edge_index[1].astype(jnp.int32)
    pad_src = (jnp.arange(NPAD, dtype=jnp.int32) * 97) % N
    pad_dst = N + (jnp.arange(NPAD, dtype=jnp.int32) % NDEAD)
    srcp = jnp.concatenate([src, pad_src]).reshape(NROW, CHK)
    dstp = jnp.concatenate([dst, pad_dst]).reshape(NROW, CHK)
    W1r = W1.reshape(D_IN, NQ1, DQ).transpose(1, 0, 2)
    b1r = b1.reshape(NQ1, 1, DQ)
    W2r = W2.reshape(NQ1, DQ, NC, DQ).transpose(0, 2, 1, 3)
    b2r = b2.reshape(NC, 1, DQ)

    hist = _deg_kernel(dstp).reshape(NC, 10240)[:, :N]
    hist3 = hist.reshape(NC, N // RB, RB).transpose(1, 0, 2)
    h1s, dinv3 = _mm1(hist3, x, W1r, b1r)
    dinv = dinv3.reshape(N)
    raw1 = _agg1(h1s, srcp, dstp, dinv)
    h2s = _mm2(raw1, dinv3, W2r, b2r)
    return _agg2(h2s, srcp, dstp, dinv)
